# baseline (speedup 1.0000x reference)
import functools

import jax
import jax.numpy as jnp
from jax import lax
from jax.experimental import pallas as pl
from jax.experimental.pallas import tpu as pltpu

_MIB = 1 << 20


def _se_pipeline(x_hbm, w1_ref, w2_ref, o_hbm, x_buf, o_buf, in_sems,
                 out_sems, *, n_steps, n_slots, inv_hw):
    def start_in(step, slot):
        pltpu.make_async_copy(x_hbm.at[step], x_buf.at[slot],
                              in_sems.at[slot]).start()

    def wait_in(slot):
        pltpu.make_async_copy(x_hbm.at[0], x_buf.at[slot],
                              in_sems.at[slot]).wait()

    def start_out(step, slot):
        pltpu.make_async_copy(o_buf.at[slot], o_hbm.at[step],
                              out_sems.at[slot]).start()

    def wait_out(slot):
        pltpu.make_async_copy(o_buf.at[slot], o_hbm.at[0],
                              out_sems.at[slot]).wait()

    for p in range(min(n_slots, n_steps)):
        start_in(p, p)

    def outer_body(outer, _):
        for slot in range(n_slots):
            step = outer * n_slots + slot
            wait_in(slot)

            @pl.when(step >= n_slots)
            def _():
                wait_out(slot)

            x = x_buf[slot]
            pooled = jnp.sum(x, axis=1, keepdims=True,
                             dtype=jnp.float32) * inv_hw
            h = lax.dot_general(w1_ref[...], pooled,
                                (((1,), (0,)), ((), ())),
                                preferred_element_type=jnp.float32)
            h = jnp.maximum(h, 0.0)
            s = lax.dot_general(w2_ref[...], h,
                                (((1,), (0,)), ((), ())),
                                preferred_element_type=jnp.float32)
            s = jax.nn.sigmoid(s).astype(x.dtype)
            o_buf[slot] = x * s
            start_out(step, slot)

            @pl.when(step + n_slots < n_steps)
            def _():
                start_in(step + n_slots, slot)
        return ()

    lax.fori_loop(0, n_steps // n_slots, outer_body, ())
    for p in range(min(n_slots, n_steps)):
        wait_out(p)


def kernel(x, w1, w2):
    B, C, H, W = x.shape
    HW = H * W
    hidden = w1.shape[0]
    inv_hw = 1.0 / float(HW)

    n_slots = next(s for s in (8, 4, 2, 1) if B % s == 0)
    x3 = x.reshape(B, C, HW)

    buf_bytes = 2 * n_slots * C * HW * x.dtype.itemsize
    vmem_limit = int(min(63 * _MIB, buf_bytes + 8 * _MIB))
    out3 = pl.pallas_call(
        functools.partial(_se_pipeline, n_steps=B, n_slots=n_slots,
                          inv_hw=inv_hw),
        out_shape=jax.ShapeDtypeStruct((B, C, HW), x.dtype),
        in_specs=[
            pl.BlockSpec(memory_space=pl.ANY),
            pl.BlockSpec(memory_space=pltpu.VMEM),
            pl.BlockSpec(memory_space=pltpu.VMEM),
        ],
        out_specs=pl.BlockSpec(memory_space=pl.ANY),
        scratch_shapes=[
            pltpu.VMEM((n_slots, C, HW), x.dtype),
            pltpu.VMEM((n_slots, C, HW), x.dtype),
            pltpu.SemaphoreType.DMA((n_slots,)),
            pltpu.SemaphoreType.DMA((n_slots,)),
        ],
        compiler_params=pltpu.CompilerParams(
            vmem_limit_bytes=vmem_limit,
        ),
    )(x3, w1, w2)
    return out3.reshape(B, C, H, W)


# speedup vs baseline: 1.0087x; 1.0087x over previous
import functools

import jax
import jax.numpy as jnp
from jax import lax
from jax.experimental import pallas as pl
from jax.experimental.pallas import tpu as pltpu

_MIB = 1 << 20


def _se_pipeline(x_hbm, w1_ref, w2_ref, o_hbm, x_buf, o_buf, in_sems,
                 out_sems, *, n_steps, n_slots, inv_hw):
    def start_in(step, slot):
        pltpu.make_async_copy(x_hbm.at[step], x_buf.at[slot],
                              in_sems.at[slot]).start()

    def wait_in(slot):
        pltpu.make_async_copy(x_hbm.at[0], x_buf.at[slot],
                              in_sems.at[slot]).wait()

    def start_out(step, slot):
        pltpu.make_async_copy(o_buf.at[slot], o_hbm.at[step],
                              out_sems.at[slot]).start()

    def wait_out(slot):
        pltpu.make_async_copy(o_buf.at[slot], o_hbm.at[0],
                              out_sems.at[slot]).wait()

    for p in range(min(n_slots, n_steps)):
        start_in(p, p)

    def outer_body(outer, _):
        for slot in range(n_slots):
            step = outer * n_slots + slot
            wait_in(slot)

            @pl.when(step >= n_slots)
            def _():
                wait_out(slot)

            x = x_buf[slot]
            pooled = jnp.sum(x, axis=1, keepdims=True,
                             dtype=jnp.float32) * inv_hw
            h = lax.dot_general(w1_ref[...], pooled,
                                (((1,), (0,)), ((), ())),
                                preferred_element_type=jnp.float32)
            h = jnp.maximum(h, 0.0)
            s = lax.dot_general(w2_ref[...], h,
                                (((1,), (0,)), ((), ())),
                                preferred_element_type=jnp.float32)
            s = jax.nn.sigmoid(s).astype(x.dtype)
            o_buf[slot] = x * s
            start_out(step, slot)

            @pl.when(step + n_slots < n_steps)
            def _():
                start_in(step + n_slots, slot)
        return ()

    lax.fori_loop(0, n_steps // n_slots, outer_body, ())
    for p in range(min(n_slots, n_steps)):
        wait_out(p)


def kernel(x, w1, w2):
    B, C, H, W = x.shape
    HW = H * W
    hidden = w1.shape[0]
    inv_hw = 1.0 / float(HW)

    n_slots = next(s for s in (4, 2, 1) if B % s == 0)
    x3 = x.reshape(B, C, HW)

    buf_bytes = 2 * n_slots * C * HW * x.dtype.itemsize
    vmem_limit = int(min(63 * _MIB, buf_bytes + 8 * _MIB))
    out3 = pl.pallas_call(
        functools.partial(_se_pipeline, n_steps=B, n_slots=n_slots,
                          inv_hw=inv_hw),
        out_shape=jax.ShapeDtypeStruct((B, C, HW), x.dtype),
        in_specs=[
            pl.BlockSpec(memory_space=pl.ANY),
            pl.BlockSpec(memory_space=pltpu.VMEM),
            pl.BlockSpec(memory_space=pltpu.VMEM),
        ],
        out_specs=pl.BlockSpec(memory_space=pl.ANY),
        scratch_shapes=[
            pltpu.VMEM((n_slots, C, HW), x.dtype),
            pltpu.VMEM((n_slots, C, HW), x.dtype),
            pltpu.SemaphoreType.DMA((n_slots,)),
            pltpu.SemaphoreType.DMA((n_slots,)),
        ],
        compiler_params=pltpu.CompilerParams(
            vmem_limit_bytes=vmem_limit,
        ),
    )(x3, w1, w2)
    return out3.reshape(B, C, H, W)
